# baseline (speedup 1.0000x reference)
import functools

import jax
import jax.numpy as jnp
from jax import lax
from jax.experimental import pallas as pl
from jax.experimental.pallas import tpu as pltpu
from jax.experimental.pallas import tpu_sc as plsc


def kernel(x, index, logits_mem):
    del x
    M, D = logits_mem.shape
    B = index.shape[0]

    info = plsc.get_sparse_core_info()
    NC, NS, L = info.num_cores, info.num_subcores, info.num_lanes
    NW = NC * NS
    assert B % NW == 0
    b_per_w = B // NW
    M0 = (M // 128) * 128
    NG = (D + L - 1) // L
    RB = 1008

    order = jnp.argsort(index)
    sorted_idx = jnp.take(index, order)
    tail_panel = jnp.pad(
        lax.slice(logits_mem, (M0, 0), (M, D)).T, ((0, 0), (0, 128 - (M - M0))))

    mesh = plsc.VectorSubcoreMesh(core_axis_name="c", subcore_axis_name="s")

    @functools.partial(
        pl.kernel,
        mesh=mesh,
        compiler_params=pltpu.CompilerParams(needs_layout_passes=False),
        out_type=jax.ShapeDtypeStruct((B * D,), jnp.float32),
        scratch_types=[
            pltpu.VMEM((b_per_w,), jnp.int32),
            pltpu.VMEM((b_per_w,), jnp.int32),
            pltpu.VMEM((D, 128), jnp.float32),
            pltpu.VMEM((2 * RB,), jnp.float32),
            pltpu.SemaphoreType.DMA,
            pltpu.SemaphoreType.DMA,
            pltpu.SemaphoreType.DMA,
        ],
    )
    def gather_sorted(sidx_hbm, ord_hbm, p_hbm, tail_hbm, out_hbm,
                      sidx_v, ord_v, panel, ring, sem0, sem1, semp):
        wid = lax.axis_index("s") * NC + lax.axis_index("c")
        base = wid * b_per_w
        pltpu.sync_copy(sidx_hbm.at[pl.ds(base, b_per_w)], sidx_v)
        pltpu.sync_copy(ord_hbm.at[pl.ds(base, b_per_w)], ord_v)
        lanes = lax.broadcasted_iota(jnp.int32, (L,), 0)

        def scalar_at(ref, s):
            v = plsc.load_gather(ref, [jnp.full((L,), s, jnp.int32)])
            return jnp.max(v)

        def body(slot, c_cur):
            i = scalar_at(sidx_v, slot)
            b = scalar_at(ord_v, slot)
            c = i >> 7
            lane = i & 127

            @pl.when(c != c_cur)
            def _stage():
                @pl.when(c != M0 // 128)
                def _full():
                    off = pl.multiple_of(c * 128, 128)
                    h = (D // 2) // 8 * 8
                    cp0 = pltpu.async_copy(
                        p_hbm.at[pl.ds(0, h), pl.ds(off, 128)],
                        panel.at[pl.ds(0, h)], semp)
                    cp1 = pltpu.async_copy(
                        p_hbm.at[pl.ds(h, D - h), pl.ds(off, 128)],
                        panel.at[pl.ds(h, D - h)], semp)
                    cp0.wait()
                    cp1.wait()

                @pl.when(c == M0 // 128)
                def _tail():
                    pltpu.sync_copy(tail_hbm, panel)

            m = slot & 1
            rbase = pl.multiple_of(m * RB, 16)

            @pl.when(slot >= 2)
            def _drain():
                @pl.when(m == 0)
                def _():
                    pltpu.make_async_copy(
                        out_hbm.at[pl.ds(0, D)],
                        ring.at[pl.ds(0, D)], sem0).wait()

                @pl.when(m == 1)
                def _():
                    pltpu.make_async_copy(
                        out_hbm.at[pl.ds(0, D)],
                        ring.at[pl.ds(RB, D)], sem1).wait()

            lane_vec = jnp.full((L,), lane, jnp.int32)
            for g in range(NG):
                d0 = g * L
                d_vec = jnp.minimum(lanes + d0, D - 1)
                vals = plsc.load_gather(panel, [d_vec, lane_vec])
                ring[pl.ds(rbase + d0, L)] = vals

            @pl.when(m == 0)
            def _out0():
                pltpu.async_copy(
                    ring.at[pl.ds(0, D)],
                    out_hbm.at[pl.ds(b * D, D)], sem0)

            @pl.when(m == 1)
            def _out1():
                pltpu.async_copy(
                    ring.at[pl.ds(RB, D)],
                    out_hbm.at[pl.ds(b * D, D)], sem1)

            return c

        lax.fori_loop(0, b_per_w, body, jnp.int32(-1))
        pltpu.make_async_copy(
            out_hbm.at[pl.ds(0, D)], ring.at[pl.ds(0, D)], sem0).wait()
        pltpu.make_async_copy(
            out_hbm.at[pl.ds(0, D)], ring.at[pl.ds(RB, D)], sem1).wait()

    flat = gather_sorted(sorted_idx, order, logits_mem.T, tail_panel)
    return flat.reshape(B, D)


# speedup vs baseline: 1.0060x; 1.0060x over previous
import functools

import jax
import jax.numpy as jnp
from jax import lax
from jax.experimental import pallas as pl
from jax.experimental.pallas import tpu as pltpu
from jax.experimental.pallas import tpu_sc as plsc


def kernel(x, index, logits_mem):
    del x
    M, D = logits_mem.shape
    B = index.shape[0]

    info = plsc.get_sparse_core_info()
    NC, NS, L = info.num_cores, info.num_subcores, info.num_lanes
    NW = NC * NS
    assert B % NW == 0
    b_per_w = B // NW
    M0 = (M // 128) * 128
    NG = (D + L - 1) // L
    RB = 1008

    order = jnp.argsort(index)
    sorted_idx = jnp.take(index, order)
    tail_panel = jnp.pad(
        lax.slice(logits_mem, (M0, 0), (M, D)).T, ((0, 0), (0, 128 - (M - M0))))

    mesh = plsc.VectorSubcoreMesh(core_axis_name="c", subcore_axis_name="s")

    @functools.partial(
        pl.kernel,
        mesh=mesh,
        compiler_params=pltpu.CompilerParams(needs_layout_passes=False),
        out_type=jax.ShapeDtypeStruct((B * D,), jnp.float32),
        scratch_types=[
            pltpu.VMEM((b_per_w,), jnp.int32),
            pltpu.VMEM((b_per_w,), jnp.int32),
            pltpu.VMEM((D, 128), jnp.float32),
            pltpu.VMEM((2 * RB,), jnp.float32),
            pltpu.SemaphoreType.DMA,
            pltpu.SemaphoreType.DMA,
        ],
    )
    def gather_sorted(sidx_hbm, ord_hbm, p_hbm, tail_hbm, out_hbm,
                      sidx_v, ord_v, panel, ring, sem0, sem1):
        wid = lax.axis_index("s") * NC + lax.axis_index("c")
        base = wid * b_per_w
        pltpu.sync_copy(sidx_hbm.at[pl.ds(base, b_per_w)], sidx_v)
        pltpu.sync_copy(ord_hbm.at[pl.ds(base, b_per_w)], ord_v)
        lanes = lax.broadcasted_iota(jnp.int32, (L,), 0)

        def scalar_at(ref, s):
            v = plsc.load_gather(ref, [jnp.full((L,), s, jnp.int32)])
            return jnp.max(v)

        def body(slot, c_cur):
            i = scalar_at(sidx_v, slot)
            b = scalar_at(ord_v, slot)
            c = i >> 7
            lane = i & 127

            @pl.when(c != c_cur)
            def _stage():
                @pl.when(c != M0 // 128)
                def _full():
                    off = pl.multiple_of(c * 128, 128)
                    pltpu.sync_copy(p_hbm.at[:, pl.ds(off, 128)], panel)

                @pl.when(c == M0 // 128)
                def _tail():
                    pltpu.sync_copy(tail_hbm, panel)

            m = slot & 1
            rbase = pl.multiple_of(m * RB, 16)

            @pl.when(slot >= 2)
            def _drain():
                @pl.when(m == 0)
                def _():
                    pltpu.make_async_copy(
                        out_hbm.at[pl.ds(0, D)],
                        ring.at[pl.ds(0, D)], sem0).wait()

                @pl.when(m == 1)
                def _():
                    pltpu.make_async_copy(
                        out_hbm.at[pl.ds(0, D)],
                        ring.at[pl.ds(RB, D)], sem1).wait()

            lane_vec = jnp.full((L,), lane, jnp.int32)
            for g in range(NG):
                d0 = g * L
                d_vec = jnp.minimum(lanes + d0, D - 1)
                vals = plsc.load_gather(panel, [d_vec, lane_vec])
                ring[pl.ds(rbase + d0, L)] = vals

            @pl.when(m == 0)
            def _out0():
                pltpu.async_copy(
                    ring.at[pl.ds(0, D)],
                    out_hbm.at[pl.ds(b * D, D)], sem0)

            @pl.when(m == 1)
            def _out1():
                pltpu.async_copy(
                    ring.at[pl.ds(RB, D)],
                    out_hbm.at[pl.ds(b * D, D)], sem1)

            return c

        lax.fori_loop(0, b_per_w, body, jnp.int32(-1))
        pltpu.make_async_copy(
            out_hbm.at[pl.ds(0, D)], ring.at[pl.ds(0, D)], sem0).wait()
        pltpu.make_async_copy(
            out_hbm.at[pl.ds(0, D)], ring.at[pl.ds(RB, D)], sem1).wait()

    flat = gather_sorted(sorted_idx, order, logits_mem.T, tail_panel)
    return flat.reshape(B, D)
